# baseline (speedup 1.0000x reference)
import functools

import numpy as np
import jax
import jax.numpy as jnp
from jax import lax
from jax.experimental import pallas as pl
from jax.experimental.pallas import tpu as pltpu
from jax.experimental.pallas import tpu_sc as plsc

_N = 50000
_D = 512
_K = 25000
_NS = 16
_CHUNK = 3136
_NV = _CHUNK // 16
_NPAD = _NS * _CHUNK
_R = 1024
_RV = _R // 16
_NPASS = 3
_GB = 784
_INV_N = np.float32(2e-05)
_EPS = np.float32(1e-08)
_MAXKEY = np.int32(0x3FFFFFFF)

_mesh = plsc.VectorSubcoreMesh(core_axis_name="c", subcore_axis_name="s")
_sc_params = pltpu.CompilerParams(needs_layout_passes=False)


def _score_body(x_ref, w_ref, o_ref):
    o_ref[...] = jnp.dot(x_ref[...], w_ref[...])


def _scores(x, wn):
    blk = 10000
    return pl.pallas_call(
        _score_body,
        grid=(_N // blk,),
        in_specs=[pl.BlockSpec((blk, _D), lambda i: (i, 0)),
                  pl.BlockSpec((_D, 1), lambda i: (0, 0))],
        out_specs=pl.BlockSpec((blk, 1), lambda i: (i, 0)),
        out_shape=jax.ShapeDtypeStruct((_N, 1), jnp.float32),
    )(x, wn)


def _sig_body(s_ref, m_ref, se_ref, sig_ref, key_ref):
    s = s_ref[...]
    z = (s - m_ref[...]) / se_ref[...]
    sig = 1.0 / (jnp.exp(-z) + 1.0)
    sig_ref[...] = sig
    key_ref[...] = _MAXKEY - lax.bitcast_convert_type(sig, jnp.int32)


def _sig_keys(s, mean, stdeps):
    return pl.pallas_call(
        _sig_body,
        out_shape=[jax.ShapeDtypeStruct((_N,), jnp.float32),
                   jax.ShapeDtypeStruct((_N,), jnp.int32)],
    )(s, mean.reshape(1), stdeps.reshape(1))


def _loss_body(sig_ref, v_ref, o_ref):
    s = sig_ref[...]
    v = v_ref[...]
    lt = jnp.log(s + _EPS)
    lb = jnp.log(1.0 - s + _EPS)
    gt = s > v
    lt_sum = jnp.sum(jnp.where(gt, lt, 0.0))
    lb_sum = jnp.sum(jnp.where(s < v, lb, 0.0))
    c_gt = jnp.sum(jnp.where(gt, 1.0, 0.0))
    c_eq = jnp.sum(jnp.where(s == v, 1.0, 0.0))
    n_top_eq = _K - c_gt
    n_rest_eq = c_eq - n_top_eq
    vs = jnp.sum(v)
    tot = (lt_sum + lb_sum + n_top_eq * jnp.log(vs + _EPS)
           + n_rest_eq * jnp.log(1.0 - vs + _EPS))
    o_ref[...] = jnp.reshape(-tot / _N, (1, 1))


def _loss(sig, vstar):
    return pl.pallas_call(
        _loss_body,
        out_shape=jax.ShapeDtypeStruct((1, 1), jnp.float32),
    )(sig, vstar)


@functools.partial(
    pl.kernel,
    out_type=[jax.ShapeDtypeStruct((_NPAD // 2,), jnp.int32),
              jax.ShapeDtypeStruct((_NPAD // 2,), jnp.float32)],
    mesh=_mesh,
    scratch_types=[
        pltpu.VMEM((_CHUNK,), jnp.int32),
        pltpu.VMEM((_CHUNK,), jnp.int32),
        pltpu.VMEM((_CHUNK,), jnp.int32),
        pltpu.VMEM((_CHUNK,), jnp.int32),
        pltpu.VMEM((_CHUNK,), jnp.int32),
        pltpu.VMEM((_CHUNK,), jnp.int32),
        pltpu.VMEM((_CHUNK,), jnp.float32),
        pltpu.VMEM((_R,), jnp.int32),
        pltpu.VMEM((_NS * _R,), jnp.int32),
        pltpu.VMEM((_R,), jnp.int32),
        pltpu.VMEM((_R,), jnp.int32),
        pltpu.VMEM((_R,), jnp.int32),
        pltpu.VMEM_SHARED((_NPAD,), jnp.int32),
        pltpu.VMEM_SHARED((_NPAD,), jnp.int32),
        pltpu.VMEM_SHARED((_NPAD,), jnp.int32),
        pltpu.VMEM_SHARED((_NPAD,), jnp.int32),
        pltpu.VMEM_SHARED((_NS * _R,), jnp.int32),
    ],
    compiler_params=_sc_params,
)
def _sort_kernel(keys_hbm, sidx_hbm, ssig_hbm, ck, ci, cd, cocc, clast, cdest,
                 cs, hl, ha, off, Hb, Pb, kA, kB, iA, iB, hall):
    cid = lax.axis_index("c")
    sid = lax.axis_index("s")
    base = sid * _CHUNK

    @pl.when(cid == 0)
    def _run():
        for p in range(_NPASS):
            shift = 10 * p
            ksrc, kdst = (kA, kB) if p % 2 == 0 else (kB, kA)
            isrc, idst = (iA, iB) if p % 2 == 0 else (iB, iA)

            if p == 0:
                pltpu.sync_copy(keys_hbm.at[pl.ds(base, _CHUNK)], ck)

                def _init_idx(i, _):
                    ci[pl.ds(i * 16, 16)] = (base + i * 16
                                             + lax.iota(jnp.int32, 16))
                    return 0
                lax.fori_loop(0, _NV, _init_idx, 0, unroll=4)
            else:
                pltpu.sync_copy(ksrc.at[pl.ds(base, _CHUNK)], ck)
                pltpu.sync_copy(isrc.at[pl.ds(base, _CHUNK)], ci)

            def _zero(i, _):
                hl[pl.ds(i * 16, 16)] = jnp.zeros((16,), jnp.int32)
                return 0
            lax.fori_loop(0, _RV, _zero, 0)

            def _hist(i, _):
                sl = pl.ds(i * 16, 16)
                d = lax.shift_right_logical(ck[sl], shift) & (_R - 1)
                occ, last = plsc.scan_count(d)
                occ = occ.astype(jnp.int32)
                cd[sl] = d
                cocc[sl] = occ
                clast[sl] = jnp.where(last, 1, 0)
                plsc.addupdate_scatter(hl, [d], occ, mask=last)
                return 0
            lax.fori_loop(0, _NV, _hist, 0, unroll=4)

            pltpu.sync_copy(hl, hall.at[pl.ds(sid * _R, _R)])
            plsc.subcore_barrier()
            pltpu.sync_copy(hall, ha)

            def _counts(vb, _):
                accH = jnp.zeros((16,), jnp.int32)
                accP = jnp.zeros((16,), jnp.int32)
                for t in range(_NS):
                    v = ha[pl.ds(t * _R + vb * 16, 16)]
                    accH = accH + v
                    accP = accP + jnp.where(t < sid, v, 0)
                Hb[pl.ds(vb * 16, 16)] = accH
                Pb[pl.ds(vb * 16, 16)] = accP
                return 0
            lax.fori_loop(0, _RV, _counts, 0, unroll=2)

            def _scan(vb, carry):
                sl = pl.ds(vb * 16, 16)
                v = Hb[sl]
                csum = plsc.cumsum(v)
                off[sl] = (csum - v) + carry + Pb[sl]
                return carry + jnp.sum(v)
            lax.fori_loop(0, _RV, _scan, jnp.int32(0))

            def _scatter(i, _):
                sl = pl.ds(i * 16, 16)
                d = cd[sl]
                occ = cocc[sl]
                lastm = clast[sl] == 1
                basev = plsc.load_gather(off, [d])
                cdest[sl] = basev + occ - 1
                plsc.addupdate_scatter(off, [d], occ, mask=lastm)
                return 0
            lax.fori_loop(0, _NV, _scatter, 0, unroll=4)

            if p < _NPASS - 1:
                pltpu.sync_copy(ck, kdst.at[cdest])
                pltpu.sync_copy(ci, idst.at[cdest])
                plsc.subcore_barrier()
            else:
                pltpu.sync_copy(ci, idst.at[cdest])
                pltpu.sync_copy(ck, kdst.at[cdest])
                plsc.subcore_barrier()

                @pl.when(sid < _NS // 2)
                def _():
                    pltpu.sync_copy(idst.at[pl.ds(base, _CHUNK)], ci)
                    pltpu.sync_copy(kdst.at[pl.ds(base, _CHUNK)], ck)

                    def _unkey(i, _):
                        sl = pl.ds(i * 16, 16)
                        cs[sl] = plsc.bitcast(_MAXKEY - ck[sl], jnp.float32)
                        return 0
                    lax.fori_loop(0, _NV, _unkey, 0, unroll=4)
                    pltpu.sync_copy(ci, sidx_hbm.at[pl.ds(base, _CHUNK)])
                    pltpu.sync_copy(cs, ssig_hbm.at[pl.ds(base, _CHUNK)])


_WV = 112
_NWAVE = _GB // _WV


@functools.partial(
    pl.kernel,
    out_type=jax.ShapeDtypeStruct((_K, _D), jnp.float32),
    mesh=_mesh,
    scratch_types=[
        pltpu.VMEM((_GB,), jnp.int32),
        pltpu.VMEM((_GB,), jnp.float32),
        pltpu.VMEM((_WV, _D), jnp.float32),
        pltpu.VMEM((_WV, _D), jnp.float32),
        pltpu.VMEM((8, _D), jnp.float32),
        pltpu.SemaphoreType.DMA,
        pltpu.SemaphoreType.DMA,
        pltpu.SemaphoreType.DMA,
        pltpu.SemaphoreType.DMA,
        pltpu.SemaphoreType.DMA,
    ],
    compiler_params=_sc_params,
)
def _gather_kernel(x_hbm, ssig_hbm, sidx_hbm, out_hbm, idxv, sigv,
                   bA, bB, bT, sem0, sem1, sems0, sems1, semt):
    cid = lax.axis_index("c")
    sid = lax.axis_index("s")
    wid = sid * 2 + cid
    base = wid * _GB
    cnt = jnp.minimum(_GB, _K - base)
    nw = cnt // _WV
    ntail = (cnt - nw * _WV) // 8

    pltpu.sync_copy(sidx_hbm.at[pl.ds(base, _GB)], idxv)
    pltpu.sync_copy(ssig_hbm.at[pl.ds(base, _GB)], sigv)

    def _scale(buf, row0, nrows):
        def _row(j, _):
            s = plsc.load_gather(sigv, [jnp.full((16,), row0 + j, jnp.int32)])
            for v in range(_D // 16):
                sl = pl.ds(v * 16, 16)
                buf[j, sl] = buf[j, sl] * s
            return 0
        lax.fori_loop(0, nrows, _row, 0)

    bufs = (bA, bB)
    gh = [pltpu.make_async_copy(
        x_hbm.at[idxv.at[pl.ds(wv * _WV, _WV)]], bufs[wv % 2],
        (sem0, sem1)[wv % 2]) for wv in range(_NWAVE)]
    sh = [pltpu.make_async_copy(
        bufs[wv % 2], out_hbm.at[pl.ds(base + wv * _WV, _WV), :],
        (sems0, sems1)[wv % 2]) for wv in range(_NWAVE)]

    for wv in range(2):
        @pl.when(wv < nw)
        def _():
            gh[wv].start()

    for wv in range(_NWAVE):
        buf = bufs[wv % 2]

        @pl.when(wv < nw)
        def _():
            gh[wv].wait()
            _scale(buf, wv * _WV, _WV)
            sh[wv].start()

        if wv + 2 < _NWAVE:
            @pl.when(wv + 2 < nw)
            def _():
                sh[wv].wait()
                gh[wv + 2].start()

    for wv in range(_NWAVE):
        @pl.when((wv < nw) & (wv + 2 >= nw))
        def _():
            sh[wv].wait()

    for t in range(3):
        @pl.when(t < ntail)
        def _():
            r0 = nw * _WV + t * 8
            pltpu.async_copy(
                x_hbm.at[idxv.at[pl.ds(r0, 8)]], bT, semt).wait()
            _scale(bT, r0, 8)
            pltpu.sync_copy(bT, out_hbm.at[pl.ds(base + r0, 8), :])


def kernel(lw_matrix_hidden_state_last, trainable_vector_pooling):
    x = lw_matrix_hidden_state_last
    w = trainable_vector_pooling

    norm2 = jnp.linalg.norm(w)
    wn = w / (norm2 + _EPS)
    s = _scores(x, wn)[:, 0]

    sb = lax.optimization_barrier(s)
    total = jnp.sum(sb)
    total = lax.optimization_barrier(total)
    mean = total * _INV_N
    mean = lax.optimization_barrier(mean)
    c = sb - mean
    var_sum = jnp.sum(c * c)
    var_sum = lax.optimization_barrier(var_sum)
    var = var_sum * _INV_N
    stdeps = jnp.sqrt(var) + _EPS

    sig, keys = _sig_keys(s, mean, stdeps)
    keys_pad = jnp.concatenate(
        [keys, jnp.full((_NPAD - _N,), _MAXKEY, jnp.int32)])

    sidx, ssig = _sort_kernel(keys_pad)
    new_x = _gather_kernel(x, ssig, sidx)
    vstar = ssig[_K - 1].reshape(1)
    pool_loss = _loss(sig, vstar)[0, 0]
    return (new_x, pool_loss)


# speedup vs baseline: 1.0120x; 1.0120x over previous
import functools

import numpy as np
import jax
import jax.numpy as jnp
from jax import lax
from jax.experimental import pallas as pl
from jax.experimental.pallas import tpu as pltpu
from jax.experimental.pallas import tpu_sc as plsc

_N = 50000
_D = 512
_K = 25000
_NS = 16
_CHUNK = 3136
_NV = _CHUNK // 16
_NPAD = _NS * _CHUNK
_R = 1024
_RV = _R // 16
_NPASS = 3
_GB = 784
_INV_N = np.float32(2e-05)
_EPS = np.float32(1e-08)
_MAXKEY = np.int32(0x3FFFFFFF)

_mesh = plsc.VectorSubcoreMesh(core_axis_name="c", subcore_axis_name="s")
_sc_params = pltpu.CompilerParams(needs_layout_passes=False)


def _score_body(x_ref, w_ref, o_ref):
    o_ref[...] = jnp.dot(x_ref[...], w_ref[...])


def _scores(x, wn):
    blk = 5000
    return pl.pallas_call(
        _score_body,
        grid=(_N // blk,),
        in_specs=[pl.BlockSpec((blk, _D), lambda i: (i, 0)),
                  pl.BlockSpec((_D, 1), lambda i: (0, 0))],
        out_specs=pl.BlockSpec((blk, 1), lambda i: (i, 0)),
        out_shape=jax.ShapeDtypeStruct((_N, 1), jnp.float32),
    )(x, wn)


def _sig_body(s_ref, m_ref, se_ref, sig_ref, key_ref):
    s = s_ref[...]
    z = (s - m_ref[...]) / se_ref[...]
    sig = 1.0 / (jnp.exp(-z) + 1.0)
    sig_ref[...] = sig
    key_ref[...] = _MAXKEY - lax.bitcast_convert_type(sig, jnp.int32)


def _sig_keys(s, mean, stdeps):
    return pl.pallas_call(
        _sig_body,
        out_shape=[jax.ShapeDtypeStruct((_N,), jnp.float32),
                   jax.ShapeDtypeStruct((_N,), jnp.int32)],
    )(s, mean.reshape(1), stdeps.reshape(1))


def _loss_body(sig_ref, v_ref, o_ref):
    s = sig_ref[...]
    v = v_ref[...]
    lt = jnp.log(s + _EPS)
    lb = jnp.log(1.0 - s + _EPS)
    gt = s > v
    lt_sum = jnp.sum(jnp.where(gt, lt, 0.0))
    lb_sum = jnp.sum(jnp.where(s < v, lb, 0.0))
    c_gt = jnp.sum(jnp.where(gt, 1.0, 0.0))
    c_eq = jnp.sum(jnp.where(s == v, 1.0, 0.0))
    n_top_eq = _K - c_gt
    n_rest_eq = c_eq - n_top_eq
    vs = jnp.sum(v)
    tot = (lt_sum + lb_sum + n_top_eq * jnp.log(vs + _EPS)
           + n_rest_eq * jnp.log(1.0 - vs + _EPS))
    o_ref[...] = jnp.reshape(-tot / _N, (1, 1))


def _loss(sig, vstar):
    return pl.pallas_call(
        _loss_body,
        out_shape=jax.ShapeDtypeStruct((1, 1), jnp.float32),
    )(sig, vstar)


@functools.partial(
    pl.kernel,
    out_type=[jax.ShapeDtypeStruct((_NPAD // 2,), jnp.int32),
              jax.ShapeDtypeStruct((_NPAD // 2,), jnp.float32)],
    mesh=_mesh,
    scratch_types=[
        pltpu.VMEM((_CHUNK,), jnp.int32),
        pltpu.VMEM((_CHUNK,), jnp.int32),
        pltpu.VMEM((_CHUNK,), jnp.int32),
        pltpu.VMEM((_CHUNK,), jnp.int32),
        pltpu.VMEM((_CHUNK,), jnp.int32),
        pltpu.VMEM((_CHUNK,), jnp.int32),
        pltpu.VMEM((_CHUNK,), jnp.float32),
        pltpu.VMEM((_R,), jnp.int32),
        pltpu.VMEM((_NS * _R,), jnp.int32),
        pltpu.VMEM((_R,), jnp.int32),
        pltpu.VMEM((_R,), jnp.int32),
        pltpu.VMEM((_R,), jnp.int32),
        pltpu.VMEM_SHARED((_NPAD,), jnp.int32),
        pltpu.VMEM_SHARED((_NPAD,), jnp.int32),
        pltpu.VMEM_SHARED((_NPAD,), jnp.int32),
        pltpu.VMEM_SHARED((_NPAD,), jnp.int32),
        pltpu.VMEM_SHARED((_NS * _R,), jnp.int32),
    ],
    compiler_params=_sc_params,
)
def _sort_kernel(keys_hbm, sidx_hbm, ssig_hbm, ck, ci, cd, cocc, clast, cdest,
                 cs, hl, ha, off, Hb, Pb, kA, kB, iA, iB, hall):
    cid = lax.axis_index("c")
    sid = lax.axis_index("s")
    base = sid * _CHUNK

    @pl.when(cid == 0)
    def _run():
        for p in range(_NPASS):
            shift = 10 * p
            ksrc, kdst = (kA, kB) if p % 2 == 0 else (kB, kA)
            isrc, idst = (iA, iB) if p % 2 == 0 else (iB, iA)

            if p == 0:
                pltpu.sync_copy(keys_hbm.at[pl.ds(base, _CHUNK)], ck)

                def _init_idx(i, _):
                    ci[pl.ds(i * 16, 16)] = (base + i * 16
                                             + lax.iota(jnp.int32, 16))
                    return 0
                lax.fori_loop(0, _NV, _init_idx, 0, unroll=4)
            else:
                pltpu.sync_copy(ksrc.at[pl.ds(base, _CHUNK)], ck)
                pltpu.sync_copy(isrc.at[pl.ds(base, _CHUNK)], ci)

            def _zero(i, _):
                hl[pl.ds(i * 16, 16)] = jnp.zeros((16,), jnp.int32)
                return 0
            lax.fori_loop(0, _RV, _zero, 0)

            def _hist(i, _):
                sl = pl.ds(i * 16, 16)
                d = lax.shift_right_logical(ck[sl], shift) & (_R - 1)
                occ, last = plsc.scan_count(d)
                occ = occ.astype(jnp.int32)
                cd[sl] = d
                cocc[sl] = occ
                clast[sl] = jnp.where(last, 1, 0)
                plsc.addupdate_scatter(hl, [d], occ, mask=last)
                return 0
            lax.fori_loop(0, _NV, _hist, 0, unroll=4)

            pltpu.sync_copy(hl, hall.at[pl.ds(sid * _R, _R)])
            plsc.subcore_barrier()
            pltpu.sync_copy(hall, ha)

            def _counts(vb, _):
                accH = jnp.zeros((16,), jnp.int32)
                accP = jnp.zeros((16,), jnp.int32)
                for t in range(_NS):
                    v = ha[pl.ds(t * _R + vb * 16, 16)]
                    accH = accH + v
                    accP = accP + jnp.where(t < sid, v, 0)
                Hb[pl.ds(vb * 16, 16)] = accH
                Pb[pl.ds(vb * 16, 16)] = accP
                return 0
            lax.fori_loop(0, _RV, _counts, 0, unroll=2)

            def _scan(vb, carry):
                sl = pl.ds(vb * 16, 16)
                v = Hb[sl]
                csum = plsc.cumsum(v)
                off[sl] = (csum - v) + carry + Pb[sl]
                return carry + jnp.sum(v)
            lax.fori_loop(0, _RV, _scan, jnp.int32(0))

            def _scatter(i, _):
                sl = pl.ds(i * 16, 16)
                d = cd[sl]
                occ = cocc[sl]
                lastm = clast[sl] == 1
                basev = plsc.load_gather(off, [d])
                cdest[sl] = basev + occ - 1
                plsc.addupdate_scatter(off, [d], occ, mask=lastm)
                return 0
            lax.fori_loop(0, _NV, _scatter, 0, unroll=4)

            if p < _NPASS - 1:
                pltpu.sync_copy(ck, kdst.at[cdest])
                pltpu.sync_copy(ci, idst.at[cdest])
                plsc.subcore_barrier()
            else:
                pltpu.sync_copy(ci, idst.at[cdest])
                pltpu.sync_copy(ck, kdst.at[cdest])
                plsc.subcore_barrier()

                @pl.when(sid < _NS // 2)
                def _():
                    pltpu.sync_copy(idst.at[pl.ds(base, _CHUNK)], ci)
                    pltpu.sync_copy(kdst.at[pl.ds(base, _CHUNK)], ck)

                    def _unkey(i, _):
                        sl = pl.ds(i * 16, 16)
                        cs[sl] = plsc.bitcast(_MAXKEY - ck[sl], jnp.float32)
                        return 0
                    lax.fori_loop(0, _NV, _unkey, 0, unroll=4)
                    pltpu.sync_copy(ci, sidx_hbm.at[pl.ds(base, _CHUNK)])
                    pltpu.sync_copy(cs, ssig_hbm.at[pl.ds(base, _CHUNK)])


_WV = 112
_NWAVE = _GB // _WV


@functools.partial(
    pl.kernel,
    out_type=jax.ShapeDtypeStruct((_K, _D), jnp.float32),
    mesh=_mesh,
    scratch_types=[
        pltpu.VMEM((_GB,), jnp.int32),
        pltpu.VMEM((_GB,), jnp.float32),
        pltpu.VMEM((_WV, _D), jnp.float32),
        pltpu.VMEM((_WV, _D), jnp.float32),
        pltpu.VMEM((8, _D), jnp.float32),
        pltpu.SemaphoreType.DMA,
        pltpu.SemaphoreType.DMA,
        pltpu.SemaphoreType.DMA,
        pltpu.SemaphoreType.DMA,
        pltpu.SemaphoreType.DMA,
    ],
    compiler_params=_sc_params,
)
def _gather_kernel(x_hbm, ssig_hbm, sidx_hbm, out_hbm, idxv, sigv,
                   bA, bB, bT, sem0, sem1, sems0, sems1, semt):
    cid = lax.axis_index("c")
    sid = lax.axis_index("s")
    wid = sid * 2 + cid
    base = wid * _GB
    cnt = jnp.minimum(_GB, _K - base)
    nw = cnt // _WV
    ntail = (cnt - nw * _WV) // 8

    pltpu.sync_copy(sidx_hbm.at[pl.ds(base, _GB)], idxv)
    pltpu.sync_copy(ssig_hbm.at[pl.ds(base, _GB)], sigv)

    def _scale(buf, row0, nrows):
        def _row(j, _):
            s = plsc.load_gather(sigv, [jnp.full((16,), row0 + j, jnp.int32)])
            for v in range(_D // 16):
                sl = pl.ds(v * 16, 16)
                buf[j, sl] = buf[j, sl] * s
            return 0
        lax.fori_loop(0, nrows, _row, 0)

    bufs = (bA, bB)
    gh = [pltpu.make_async_copy(
        x_hbm.at[idxv.at[pl.ds(wv * _WV, _WV)]], bufs[wv % 2],
        (sem0, sem1)[wv % 2]) for wv in range(_NWAVE)]
    sh = [pltpu.make_async_copy(
        bufs[wv % 2], out_hbm.at[pl.ds(base + wv * _WV, _WV), :],
        (sems0, sems1)[wv % 2]) for wv in range(_NWAVE)]

    for wv in range(2):
        @pl.when(wv < nw)
        def _():
            gh[wv].start()

    for wv in range(_NWAVE):
        buf = bufs[wv % 2]

        @pl.when(wv < nw)
        def _():
            gh[wv].wait()
            _scale(buf, wv * _WV, _WV)
            sh[wv].start()

        if wv + 2 < _NWAVE:
            @pl.when(wv + 2 < nw)
            def _():
                sh[wv].wait()
                gh[wv + 2].start()

    for wv in range(_NWAVE):
        @pl.when((wv < nw) & (wv + 2 >= nw))
        def _():
            sh[wv].wait()

    for t in range(3):
        @pl.when(t < ntail)
        def _():
            r0 = nw * _WV + t * 8
            pltpu.async_copy(
                x_hbm.at[idxv.at[pl.ds(r0, 8)]], bT, semt).wait()
            _scale(bT, r0, 8)
            pltpu.sync_copy(bT, out_hbm.at[pl.ds(base + r0, 8), :])


def kernel(lw_matrix_hidden_state_last, trainable_vector_pooling):
    x = lw_matrix_hidden_state_last
    w = trainable_vector_pooling

    norm2 = jnp.linalg.norm(w)
    wn = w / (norm2 + _EPS)
    s = _scores(x, wn)[:, 0]

    sb = lax.optimization_barrier(s)
    total = jnp.sum(sb)
    total = lax.optimization_barrier(total)
    mean = total * _INV_N
    mean = lax.optimization_barrier(mean)
    c = sb - mean
    var_sum = jnp.sum(c * c)
    var_sum = lax.optimization_barrier(var_sum)
    var = var_sum * _INV_N
    stdeps = jnp.sqrt(var) + _EPS

    sig, keys = _sig_keys(s, mean, stdeps)
    keys_pad = jnp.concatenate(
        [keys, jnp.full((_NPAD - _N,), _MAXKEY, jnp.int32)])

    sidx, ssig = _sort_kernel(keys_pad)
    new_x = _gather_kernel(x, ssig, sidx)
    vstar = ssig[_K - 1].reshape(1)
    pool_loss = _loss(sig, vstar)[0, 0]
    return (new_x, pool_loss)


# speedup vs baseline: 1.0307x; 1.0185x over previous
import functools

import numpy as np
import jax
import jax.numpy as jnp
from jax import lax
from jax.experimental import pallas as pl
from jax.experimental.pallas import tpu as pltpu
from jax.experimental.pallas import tpu_sc as plsc

_N = 50000
_D = 512
_K = 25000
_NS = 16
_CHUNK = 3136
_NV = _CHUNK // 16
_NPAD = _NS * _CHUNK
_R = 1024
_RV = _R // 16
_NPASS = 3
_GB = 784
_INV_N = np.float32(2e-05)
_EPS = np.float32(1e-08)
_MAXKEY = np.int32(0x3FFFFFFF)

_mesh = plsc.VectorSubcoreMesh(core_axis_name="c", subcore_axis_name="s")
_sc_params = pltpu.CompilerParams(needs_layout_passes=False)


def _score_body(x_ref, w_ref, o_ref):
    o_ref[...] = jnp.dot(x_ref[...], w_ref[...])


def _scores(x, wn):
    blk = 5000
    return pl.pallas_call(
        _score_body,
        grid=(_N // blk,),
        in_specs=[pl.BlockSpec((blk, _D), lambda i: (i, 0)),
                  pl.BlockSpec((_D, 1), lambda i: (0, 0))],
        out_specs=pl.BlockSpec((blk, 1), lambda i: (i, 0)),
        out_shape=jax.ShapeDtypeStruct((_N, 1), jnp.float32),
    )(x, wn)


def _sig_body(s_ref, m_ref, se_ref, sig_ref, key_ref):
    s = s_ref[...]
    z = (s - m_ref[...]) / se_ref[...]
    sig = 1.0 / (jnp.exp(-z) + 1.0)
    sig_ref[...] = sig
    key_ref[pl.ds(0, _N)] = _MAXKEY - lax.bitcast_convert_type(sig, jnp.int32)
    key_ref[pl.ds(_N, _NPAD - _N)] = jnp.full((_NPAD - _N,), _MAXKEY,
                                              jnp.int32)


def _sig_keys(s, mean, stdeps):
    return pl.pallas_call(
        _sig_body,
        out_shape=[jax.ShapeDtypeStruct((_N,), jnp.float32),
                   jax.ShapeDtypeStruct((_NPAD,), jnp.int32)],
    )(s, mean.reshape(1), stdeps.reshape(1))


def _loss_body(sig_ref, v_ref, o_ref):
    s = sig_ref[...]
    v = v_ref[...]
    lt = jnp.log(s + _EPS)
    lb = jnp.log(1.0 - s + _EPS)
    gt = s > v
    lt_sum = jnp.sum(jnp.where(gt, lt, 0.0))
    lb_sum = jnp.sum(jnp.where(s < v, lb, 0.0))
    c_gt = jnp.sum(jnp.where(gt, 1.0, 0.0))
    c_eq = jnp.sum(jnp.where(s == v, 1.0, 0.0))
    n_top_eq = _K - c_gt
    n_rest_eq = c_eq - n_top_eq
    vs = jnp.sum(v)
    tot = (lt_sum + lb_sum + n_top_eq * jnp.log(vs + _EPS)
           + n_rest_eq * jnp.log(1.0 - vs + _EPS))
    o_ref[...] = jnp.reshape(-tot / _N, (1, 1))


def _loss(sig, vstar):
    return pl.pallas_call(
        _loss_body,
        out_shape=jax.ShapeDtypeStruct((1, 1), jnp.float32),
    )(sig, vstar)


@functools.partial(
    pl.kernel,
    out_type=[jax.ShapeDtypeStruct((_NPAD // 2,), jnp.int32),
              jax.ShapeDtypeStruct((_NPAD // 2,), jnp.float32)],
    mesh=_mesh,
    scratch_types=[
        pltpu.VMEM((_CHUNK,), jnp.int32),
        pltpu.VMEM((_CHUNK,), jnp.int32),
        pltpu.VMEM((_CHUNK,), jnp.int32),
        pltpu.VMEM((_CHUNK,), jnp.int32),
        pltpu.VMEM((_CHUNK,), jnp.int32),
        pltpu.VMEM((_CHUNK,), jnp.int32),
        pltpu.VMEM((_CHUNK,), jnp.float32),
        pltpu.VMEM((_R,), jnp.int32),
        pltpu.VMEM((_NS * _R,), jnp.int32),
        pltpu.VMEM((_R,), jnp.int32),
        pltpu.VMEM((_R,), jnp.int32),
        pltpu.VMEM((_R,), jnp.int32),
        pltpu.VMEM_SHARED((_NPAD,), jnp.int32),
        pltpu.VMEM_SHARED((_NPAD,), jnp.int32),
        pltpu.VMEM_SHARED((_NPAD,), jnp.int32),
        pltpu.VMEM_SHARED((_NPAD,), jnp.int32),
        pltpu.VMEM_SHARED((_NS * _R,), jnp.int32),
    ],
    compiler_params=_sc_params,
)
def _sort_kernel(keys_hbm, sidx_hbm, ssig_hbm, ck, ci, cd, cocc, clast, cdest,
                 cs, hl, ha, off, Hb, Pb, kA, kB, iA, iB, hall):
    cid = lax.axis_index("c")
    sid = lax.axis_index("s")
    base = sid * _CHUNK

    @pl.when(cid == 0)
    def _run():
        for p in range(_NPASS):
            shift = 10 * p
            ksrc, kdst = (kA, kB) if p % 2 == 0 else (kB, kA)
            isrc, idst = (iA, iB) if p % 2 == 0 else (iB, iA)

            if p == 0:
                pltpu.sync_copy(keys_hbm.at[pl.ds(base, _CHUNK)], ck)

                def _init_idx(i, _):
                    ci[pl.ds(i * 16, 16)] = (base + i * 16
                                             + lax.iota(jnp.int32, 16))
                    return 0
                lax.fori_loop(0, _NV, _init_idx, 0, unroll=4)
            else:
                pltpu.sync_copy(ksrc.at[pl.ds(base, _CHUNK)], ck)
                pltpu.sync_copy(isrc.at[pl.ds(base, _CHUNK)], ci)

            def _zero(i, _):
                hl[pl.ds(i * 16, 16)] = jnp.zeros((16,), jnp.int32)
                return 0
            lax.fori_loop(0, _RV, _zero, 0)

            def _hist(i, _):
                sl = pl.ds(i * 16, 16)
                d = lax.shift_right_logical(ck[sl], shift) & (_R - 1)
                occ, last = plsc.scan_count(d)
                occ = occ.astype(jnp.int32)
                cd[sl] = d
                cocc[sl] = occ
                clast[sl] = jnp.where(last, 1, 0)
                plsc.addupdate_scatter(hl, [d], occ, mask=last)
                return 0
            lax.fori_loop(0, _NV, _hist, 0, unroll=4)

            pltpu.sync_copy(hl, hall.at[pl.ds(sid * _R, _R)])
            plsc.subcore_barrier()
            pltpu.sync_copy(hall, ha)

            def _counts(vb, _):
                accH = jnp.zeros((16,), jnp.int32)
                accP = jnp.zeros((16,), jnp.int32)
                for t in range(_NS):
                    v = ha[pl.ds(t * _R + vb * 16, 16)]
                    accH = accH + v
                    accP = accP + jnp.where(t < sid, v, 0)
                Hb[pl.ds(vb * 16, 16)] = accH
                Pb[pl.ds(vb * 16, 16)] = accP
                return 0
            lax.fori_loop(0, _RV, _counts, 0, unroll=2)

            def _scan(vb, carry):
                sl = pl.ds(vb * 16, 16)
                v = Hb[sl]
                csum = plsc.cumsum(v)
                off[sl] = (csum - v) + carry + Pb[sl]
                return carry + jnp.sum(v)
            lax.fori_loop(0, _RV, _scan, jnp.int32(0))

            def _scatter(i, _):
                sl = pl.ds(i * 16, 16)
                d = cd[sl]
                occ = cocc[sl]
                lastm = clast[sl] == 1
                basev = plsc.load_gather(off, [d])
                cdest[sl] = basev + occ - 1
                plsc.addupdate_scatter(off, [d], occ, mask=lastm)
                return 0
            lax.fori_loop(0, _NV, _scatter, 0, unroll=4)

            if p < _NPASS - 1:
                pltpu.sync_copy(ck, kdst.at[cdest])
                pltpu.sync_copy(ci, idst.at[cdest])
                plsc.subcore_barrier()
            else:
                pltpu.sync_copy(ci, idst.at[cdest])
                pltpu.sync_copy(ck, kdst.at[cdest])
                plsc.subcore_barrier()

                @pl.when(sid < _NS // 2)
                def _():
                    pltpu.sync_copy(idst.at[pl.ds(base, _CHUNK)], ci)
                    pltpu.sync_copy(kdst.at[pl.ds(base, _CHUNK)], ck)

                    def _unkey(i, _):
                        sl = pl.ds(i * 16, 16)
                        cs[sl] = plsc.bitcast(_MAXKEY - ck[sl], jnp.float32)
                        return 0
                    lax.fori_loop(0, _NV, _unkey, 0, unroll=4)
                    pltpu.sync_copy(ci, sidx_hbm.at[pl.ds(base, _CHUNK)])
                    pltpu.sync_copy(cs, ssig_hbm.at[pl.ds(base, _CHUNK)])


_WV = 112
_NWAVE = _GB // _WV


@functools.partial(
    pl.kernel,
    out_type=jax.ShapeDtypeStruct((_K, _D), jnp.float32),
    mesh=_mesh,
    scratch_types=[
        pltpu.VMEM((_GB,), jnp.int32),
        pltpu.VMEM((_GB,), jnp.float32),
        pltpu.VMEM((_WV, _D), jnp.float32),
        pltpu.VMEM((_WV, _D), jnp.float32),
        pltpu.VMEM((8, _D), jnp.float32),
        pltpu.SemaphoreType.DMA,
        pltpu.SemaphoreType.DMA,
        pltpu.SemaphoreType.DMA,
        pltpu.SemaphoreType.DMA,
        pltpu.SemaphoreType.DMA,
    ],
    compiler_params=_sc_params,
)
def _gather_kernel(x_hbm, ssig_hbm, sidx_hbm, out_hbm, idxv, sigv,
                   bA, bB, bT, sem0, sem1, sems0, sems1, semt):
    cid = lax.axis_index("c")
    sid = lax.axis_index("s")
    wid = sid * 2 + cid
    base = wid * _GB
    cnt = jnp.minimum(_GB, _K - base)
    nw = cnt // _WV
    ntail = (cnt - nw * _WV) // 8

    pltpu.sync_copy(sidx_hbm.at[pl.ds(base, _GB)], idxv)
    pltpu.sync_copy(ssig_hbm.at[pl.ds(base, _GB)], sigv)

    def _scale(buf, row0, nrows):
        def _row(j, _):
            s = plsc.load_gather(sigv, [jnp.full((16,), row0 + j, jnp.int32)])
            for v in range(_D // 16):
                sl = pl.ds(v * 16, 16)
                buf[j, sl] = buf[j, sl] * s
            return 0
        lax.fori_loop(0, nrows, _row, 0)

    bufs = (bA, bB)
    gh = [pltpu.make_async_copy(
        x_hbm.at[idxv.at[pl.ds(wv * _WV, _WV)]], bufs[wv % 2],
        (sem0, sem1)[wv % 2]) for wv in range(_NWAVE)]
    sh = [pltpu.make_async_copy(
        bufs[wv % 2], out_hbm.at[pl.ds(base + wv * _WV, _WV), :],
        (sems0, sems1)[wv % 2]) for wv in range(_NWAVE)]

    for wv in range(2):
        @pl.when(wv < nw)
        def _():
            gh[wv].start()

    for wv in range(_NWAVE):
        buf = bufs[wv % 2]

        @pl.when(wv < nw)
        def _():
            gh[wv].wait()
            _scale(buf, wv * _WV, _WV)
            sh[wv].start()

        if wv + 2 < _NWAVE:
            @pl.when(wv + 2 < nw)
            def _():
                sh[wv].wait()
                gh[wv + 2].start()

    for wv in range(_NWAVE):
        @pl.when((wv < nw) & (wv + 2 >= nw))
        def _():
            sh[wv].wait()

    for t in range(3):
        @pl.when(t < ntail)
        def _():
            r0 = nw * _WV + t * 8
            pltpu.async_copy(
                x_hbm.at[idxv.at[pl.ds(r0, 8)]], bT, semt).wait()
            _scale(bT, r0, 8)
            pltpu.sync_copy(bT, out_hbm.at[pl.ds(base + r0, 8), :])


def kernel(lw_matrix_hidden_state_last, trainable_vector_pooling):
    x = lw_matrix_hidden_state_last
    w = trainable_vector_pooling

    norm2 = jnp.linalg.norm(w)
    wn = w / (norm2 + _EPS)
    s = _scores(x, wn)[:, 0]

    sb = lax.optimization_barrier(s)
    total = jnp.sum(sb)
    total = lax.optimization_barrier(total)
    mean = total * _INV_N
    mean = lax.optimization_barrier(mean)
    c = sb - mean
    var_sum = jnp.sum(c * c)
    var_sum = lax.optimization_barrier(var_sum)
    var = var_sum * _INV_N
    stdeps = jnp.sqrt(var) + _EPS

    sig, keys_pad = _sig_keys(s, mean, stdeps)

    sidx, ssig = _sort_kernel(keys_pad)
    new_x = _gather_kernel(x, ssig, sidx)
    vstar = ssig[_K - 1].reshape(1)
    pool_loss = _loss(sig, vstar)[0, 0]
    return (new_x, pool_loss)
